# baseline (speedup 1.0000x reference)
import functools

import jax
import jax.numpy as jnp
from jax import lax
from jax.experimental import pallas as pl
from jax.experimental.pallas import tpu as pltpu
from jax.experimental.pallas import tpu_sc as plsc

N = 10000
E = 320000
D = 128

NC = 2
NS = 16
NW = NC * NS
N_PAD = 10240
ROWS_PER_TILE = N_PAD // NS

EDGE_B = 184
EDGE_C = 56
E_PER_W = EDGE_B * EDGE_C
E_PAD = E_PER_W * NW

DEG_B = 1288
DEG_E_PER_W = E_PER_W
DEG_CHUNKS = DEG_E_PER_W // DEG_B

_mesh = plsc.VectorSubcoreMesh(core_axis_name="c", subcore_axis_name="s")


@functools.partial(
    pl.kernel,
    out_type=jax.ShapeDtypeStruct((2 * N_PAD,), jnp.float32),
    mesh=_mesh,
    scratch_types=[
        pltpu.VMEM((DEG_B,), jnp.int32),
        pltpu.VMEM((DEG_B,), jnp.float32),
        pltpu.VMEM_SHARED((N_PAD,), jnp.float32),
    ],
)
def _sc_degree(row_hbm, zeros_hbm, ones_hbm, out_hbm, idx_v, ones_v, deg_sh):
    cid = lax.axis_index("c")
    sid = lax.axis_index("s")
    wid = sid * NC + cid
    r0 = sid * ROWS_PER_TILE
    pltpu.sync_copy(zeros_hbm.at[pl.ds(r0, ROWS_PER_TILE)],
                    deg_sh.at[pl.ds(r0, ROWS_PER_TILE)])
    pltpu.sync_copy(ones_hbm, ones_v)
    plsc.subcore_barrier()

    base = wid * DEG_E_PER_W

    @pl.loop(0, DEG_CHUNKS)
    def _(k):
        pltpu.sync_copy(row_hbm.at[pl.ds(base + k * DEG_B, DEG_B)], idx_v)
        pltpu.sync_copy(ones_v, deg_sh.at[idx_v], add=True)

    plsc.subcore_barrier()
    pltpu.sync_copy(deg_sh.at[pl.ds(r0, ROWS_PER_TILE)],
                    out_hbm.at[pl.ds(cid * N_PAD + r0, ROWS_PER_TILE)])


@functools.partial(
    pl.kernel,
    out_type=jax.ShapeDtypeStruct((2 * N_PAD, D), jnp.float32),
    mesh=_mesh,
    scratch_types=[
        pltpu.VMEM((EDGE_B,), jnp.int32),
        pltpu.VMEM((EDGE_B,), jnp.int32),
        pltpu.VMEM((EDGE_B,), jnp.int32),
        pltpu.VMEM((EDGE_B,), jnp.int32),
        pltpu.VMEM((EDGE_B,), jnp.int32),
        pltpu.VMEM((EDGE_B,), jnp.int32),
        pltpu.VMEM((EDGE_B,), jnp.int32),
        pltpu.VMEM((EDGE_B,), jnp.int32),
        pltpu.VMEM((EDGE_B, D), jnp.float32),
        pltpu.VMEM((EDGE_B, D), jnp.float32),
        pltpu.VMEM_SHARED((N_PAD, D), jnp.float32),
        pltpu.SemaphoreType.DMA,
        pltpu.SemaphoreType.DMA,
        pltpu.SemaphoreType.DMA,
        pltpu.SemaphoreType.DMA,
        pltpu.SemaphoreType.DMA,
        pltpu.SemaphoreType.DMA,
        pltpu.SemaphoreType.DMA,
        pltpu.SemaphoreType.DMA,
        pltpu.SemaphoreType.DMA,
        pltpu.SemaphoreType.DMA,
        pltpu.SemaphoreType.DMA,
        pltpu.SemaphoreType.DMA,
    ],
)
def _sc_edges(row_hbm, col_hbm, g_hbm, zeros_hbm, out_hbm,
              rb0, rb1, rb2, rb3, cb0, cb1, cb2, cb3, rows0, rows1, acc_sh,
              sr0, sr1, sr2, sr3, sc0, sc1, sc2, sc3, sg0, sg1, ss0, ss1):
    cid = lax.axis_index("c")
    sid = lax.axis_index("s")
    wid = sid * NC + cid
    r0 = sid * ROWS_PER_TILE
    pltpu.sync_copy(zeros_hbm.at[pl.ds(r0, ROWS_PER_TILE)],
                    acc_sh.at[pl.ds(r0, ROWS_PER_TILE)])
    plsc.subcore_barrier()

    base = wid * E_PER_W
    rb = (rb0, rb1, rb2, rb3)
    cb = (cb0, cb1, cb2, cb3)
    rows = (rows0, rows1)
    sr = (sr0, sr1, sr2, sr3)
    sc = (sc0, sc1, sc2, sc3)
    sg = (sg0, sg1)
    ss = (ss0, ss1)

    def start_idx(k, p4):
        off = base + k * EDGE_B
        pltpu.async_copy(row_hbm.at[pl.ds(off, EDGE_B)], rb[p4], sr[p4])
        pltpu.async_copy(col_hbm.at[pl.ds(off, EDGE_B)], cb[p4], sc[p4])

    def wait_row(k, p4):
        pltpu.make_async_copy(
            row_hbm.at[pl.ds(base + k * EDGE_B, EDGE_B)], rb[p4], sr[p4]).wait()

    def wait_col(k, p4):
        pltpu.make_async_copy(
            col_hbm.at[pl.ds(base + k * EDGE_B, EDGE_B)], cb[p4], sc[p4]).wait()

    def start_gather(p4, p2):
        return pltpu.async_copy(g_hbm.at[rb[p4]], rows[p2], sg[p2])

    def wait_gather(p4, p2):
        pltpu.make_async_copy(g_hbm.at[rb[p4]], rows[p2], sg[p2]).wait()

    def start_scatter(p2, p4):
        return pltpu.async_copy(rows[p2], acc_sh.at[cb[p4]], ss[p2], add=True)

    def wait_scatter(p2, p4):
        pltpu.make_async_copy(rows[p2], acc_sh.at[cb[p4]], ss[p2]).wait()

    start_idx(0, 0)
    start_idx(1, 1)
    start_idx(2, 2)
    start_idx(3, 3)
    wait_row(0, 0)
    start_gather(0, 0)
    wait_row(1, 1)
    start_gather(1, 1)
    wait_gather(0, 0)
    wait_col(0, 0)
    start_scatter(0, 0)

    @pl.loop(0, (EDGE_C - 4) // 4)
    def _(j):
        for p in range(4):
            k = 2 + 4 * j + p
            p2 = p % 2
            p4 = (2 + p) % 4
            wait_scatter(p2, p)
            start_idx(k + 2, p)
            wait_row(k, p4)
            start_gather(p4, p2)
            q2 = 1 - p2
            q4 = (p + 1) % 4
            wait_gather(q4, q2)
            wait_col(k - 1, q4)
            start_scatter(q2, q4)

    wait_scatter(0, 0)
    wait_row(EDGE_C - 2, 2)
    start_gather(2, 0)
    wait_gather(1, 1)
    wait_col(EDGE_C - 3, 1)
    start_scatter(1, 1)
    wait_scatter(1, 1)
    wait_row(EDGE_C - 1, 3)
    start_gather(3, 1)
    wait_gather(2, 0)
    wait_col(EDGE_C - 2, 2)
    start_scatter(0, 2)
    wait_gather(3, 1)
    wait_col(EDGE_C - 1, 3)
    start_scatter(1, 3)
    wait_scatter(0, 2)
    wait_scatter(1, 3)

    plsc.subcore_barrier()
    pltpu.sync_copy(acc_sh.at[pl.ds(r0, ROWS_PER_TILE)],
                    out_hbm.at[pl.ds(cid * N_PAD + r0, ROWS_PER_TILE)])


_BLK = 640
_GRID = -(-N // _BLK)


def _linear_body(x_ref, w_ref, b_ref, dega_ref, degb_ref, o_ref):
    h = lax.dot_general(
        x_ref[...], w_ref[...], (((1,), (1,)), ((), ())),
        preferred_element_type=jnp.float32) + b_ref[...]
    deg = dega_ref[...] + degb_ref[...] + 1.0
    o_ref[...] = h * lax.rsqrt(deg)


def _tc_linear_scale(x, W, b, deg_col):
    return pl.pallas_call(
        _linear_body,
        grid=(_GRID,),
        in_specs=[
            pl.BlockSpec((_BLK, D), lambda i: (i, 0)),
            pl.BlockSpec((D, D), lambda i: (0, 0)),
            pl.BlockSpec((1, D), lambda i: (0, 0)),
            pl.BlockSpec((_BLK, 1), lambda i: (i, 0)),
            pl.BlockSpec((_BLK, 1), lambda i: (i + N_PAD // _BLK, 0)),
        ],
        out_specs=pl.BlockSpec((_BLK, D), lambda i: (i, 0)),
        out_shape=jax.ShapeDtypeStruct((N_PAD, D), jnp.float32),
    )(x, W, b.reshape(1, D), deg_col, deg_col)


def _final_body(s0_ref, s1_ref, g_ref, dega_ref, degb_ref, x_ref,
                gamma_ref, beta_ref, o_ref):
    deg = dega_ref[...] + degb_ref[...] + 1.0
    dinv = lax.rsqrt(deg)
    t = dinv * (s0_ref[...] + s1_ref[...] + g_ref[...])
    t = jnp.maximum(t, 0.0)
    mu = jnp.mean(t, axis=-1, keepdims=True)
    var = jnp.mean((t - mu) ** 2, axis=-1, keepdims=True)
    y = (t - mu) * lax.rsqrt(var + 1e-5)
    o_ref[...] = y * gamma_ref[...] + beta_ref[...] + x_ref[...]


def _tc_final(s_p, g, deg_p, x, gamma, beta):
    return pl.pallas_call(
        _final_body,
        grid=(_GRID,),
        in_specs=[
            pl.BlockSpec((_BLK, D), lambda i: (i, 0)),
            pl.BlockSpec((_BLK, D), lambda i: (i + N_PAD // _BLK, 0)),
            pl.BlockSpec((_BLK, D), lambda i: (i, 0)),
            pl.BlockSpec((_BLK, 1), lambda i: (i, 0)),
            pl.BlockSpec((_BLK, 1), lambda i: (i + N_PAD // _BLK, 0)),
            pl.BlockSpec((_BLK, D), lambda i: (i, 0)),
            pl.BlockSpec((1, D), lambda i: (0, 0)),
            pl.BlockSpec((1, D), lambda i: (0, 0)),
        ],
        out_specs=pl.BlockSpec((_BLK, D), lambda i: (i, 0)),
        out_shape=jax.ShapeDtypeStruct((N, D), jnp.float32),
    )(s_p, s_p, g, deg_p, deg_p, x, gamma.reshape(1, D), beta.reshape(1, D))


def kernel(x, edge_index, W, b, gamma, beta):
    pad_idx = N + (jnp.arange(E_PAD - E, dtype=jnp.int32) % (N_PAD - N))
    row = jnp.concatenate([edge_index[0].astype(jnp.int32), pad_idx])
    col = jnp.concatenate([edge_index[1].astype(jnp.int32), pad_idx])
    x_pad = jnp.concatenate([x, jnp.zeros((N_PAD - N, D), jnp.float32)])

    zeros_deg = jnp.zeros((N_PAD,), jnp.float32)
    ones_upd = jnp.ones((DEG_B,), jnp.float32)
    zeros_acc = jnp.zeros((N_PAD, D), jnp.float32)

    deg_p = _sc_degree(row, zeros_deg, ones_upd)
    deg_col = deg_p.reshape(2 * N_PAD, 1)
    g = _tc_linear_scale(x_pad, W, b, deg_col)
    s_p = _sc_edges(row, col, g, zeros_acc)
    return _tc_final(s_p, g, deg_col, x, gamma, beta)


# speedup vs baseline: 1.0297x; 1.0297x over previous
import functools

import jax
import jax.numpy as jnp
from jax import lax
from jax.experimental import pallas as pl
from jax.experimental.pallas import tpu as pltpu
from jax.experimental.pallas import tpu_sc as plsc

N = 10000
E = 320000
D = 128

NC = 2
NS = 16
NW = NC * NS
N_PAD = 10240
ROWS_PER_TILE = N_PAD // NS

EDGE_B = 160
EDGE_C = 64
EDGE_C_LAST = 16
E_PER_W = EDGE_B * EDGE_C

DEG_B = 2000
DEG_E_PER_W = E // NW
DEG_CHUNKS = DEG_E_PER_W // DEG_B

_mesh = plsc.VectorSubcoreMesh(core_axis_name="c", subcore_axis_name="s")


@functools.partial(
    pl.kernel,
    out_type=jax.ShapeDtypeStruct((2 * N_PAD,), jnp.float32),
    mesh=_mesh,
    scratch_types=[
        pltpu.VMEM((DEG_B,), jnp.int32),
        pltpu.VMEM((DEG_B,), jnp.float32),
        pltpu.VMEM_SHARED((N_PAD,), jnp.float32),
    ],
)
def _sc_degree(row_hbm, zeros_hbm, ones_hbm, out_hbm, idx_v, ones_v, deg_sh):
    cid = lax.axis_index("c")
    sid = lax.axis_index("s")
    wid = sid * NC + cid
    r0 = sid * ROWS_PER_TILE
    pltpu.sync_copy(zeros_hbm.at[pl.ds(r0, ROWS_PER_TILE)],
                    deg_sh.at[pl.ds(r0, ROWS_PER_TILE)])
    pltpu.sync_copy(ones_hbm, ones_v)
    plsc.subcore_barrier()

    base = wid * DEG_E_PER_W

    @pl.loop(0, DEG_CHUNKS)
    def _(k):
        pltpu.sync_copy(row_hbm.at[pl.ds(base + k * DEG_B, DEG_B)], idx_v)
        pltpu.sync_copy(ones_v, deg_sh.at[idx_v], add=True)

    plsc.subcore_barrier()
    pltpu.sync_copy(deg_sh.at[pl.ds(r0, ROWS_PER_TILE)],
                    out_hbm.at[pl.ds(cid * N_PAD + r0, ROWS_PER_TILE)])


@functools.partial(
    pl.kernel,
    out_type=jax.ShapeDtypeStruct((2 * N_PAD, D), jnp.float32),
    mesh=_mesh,
    scratch_types=[
        pltpu.VMEM((EDGE_B,), jnp.int32),
        pltpu.VMEM((EDGE_B,), jnp.int32),
        pltpu.VMEM((EDGE_B,), jnp.int32),
        pltpu.VMEM((EDGE_B,), jnp.int32),
        pltpu.VMEM((EDGE_B,), jnp.int32),
        pltpu.VMEM((EDGE_B,), jnp.int32),
        pltpu.VMEM((EDGE_B,), jnp.int32),
        pltpu.VMEM((EDGE_B,), jnp.int32),
        pltpu.VMEM((EDGE_B, D), jnp.float32),
        pltpu.VMEM((EDGE_B, D), jnp.float32),
        pltpu.VMEM_SHARED((N_PAD, D), jnp.float32),
        pltpu.SemaphoreType.DMA,
        pltpu.SemaphoreType.DMA,
        pltpu.SemaphoreType.DMA,
        pltpu.SemaphoreType.DMA,
        pltpu.SemaphoreType.DMA,
        pltpu.SemaphoreType.DMA,
        pltpu.SemaphoreType.DMA,
        pltpu.SemaphoreType.DMA,
        pltpu.SemaphoreType.DMA,
        pltpu.SemaphoreType.DMA,
        pltpu.SemaphoreType.DMA,
        pltpu.SemaphoreType.DMA,
    ],
)
def _sc_edges(row_hbm, col_hbm, g_hbm, zeros_hbm, out_hbm,
              rb0, rb1, rb2, rb3, cb0, cb1, cb2, cb3, rows0, rows1, acc_sh,
              sr0, sr1, sr2, sr3, sc0, sc1, sc2, sc3, sg0, sg1, ss0, ss1):
    cid = lax.axis_index("c")
    sid = lax.axis_index("s")
    wid = sid * NC + cid
    r0 = sid * ROWS_PER_TILE
    base = wid * E_PER_W
    n_chunks = jnp.where(wid == NW - 1, EDGE_C_LAST, EDGE_C)
    rb = (rb0, rb1, rb2, rb3)
    cb = (cb0, cb1, cb2, cb3)
    rows = (rows0, rows1)
    sr = (sr0, sr1, sr2, sr3)
    sc = (sc0, sc1, sc2, sc3)
    sg = (sg0, sg1)
    ss = (ss0, ss1)

    def start_idx(k, p4):
        off = base + k * EDGE_B
        pltpu.async_copy(row_hbm.at[pl.ds(off, EDGE_B)], rb[p4], sr[p4])
        pltpu.async_copy(col_hbm.at[pl.ds(off, EDGE_B)], cb[p4], sc[p4])

    def wait_row(k, p4):
        pltpu.make_async_copy(
            row_hbm.at[pl.ds(base + k * EDGE_B, EDGE_B)], rb[p4], sr[p4]).wait()

    def wait_col(k, p4):
        pltpu.make_async_copy(
            col_hbm.at[pl.ds(base + k * EDGE_B, EDGE_B)], cb[p4], sc[p4]).wait()

    def start_gather(p4, p2):
        return pltpu.async_copy(g_hbm.at[rb[p4]], rows[p2], sg[p2])

    def wait_gather(p4, p2):
        pltpu.make_async_copy(g_hbm.at[rb[p4]], rows[p2], sg[p2]).wait()

    def start_scatter(p2, p4):
        return pltpu.async_copy(rows[p2], acc_sh.at[cb[p4]], ss[p2], add=True)

    def wait_scatter(p2, p4):
        pltpu.make_async_copy(rows[p2], acc_sh.at[cb[p4]], ss[p2]).wait()

    start_idx(0, 0)
    start_idx(1, 1)
    start_idx(2, 2)
    start_idx(3, 3)
    wait_row(0, 0)
    start_gather(0, 0)
    wait_row(1, 1)
    start_gather(1, 1)
    pltpu.sync_copy(zeros_hbm.at[pl.ds(r0, ROWS_PER_TILE)],
                    acc_sh.at[pl.ds(r0, ROWS_PER_TILE)])
    plsc.subcore_barrier()
    wait_gather(0, 0)
    wait_col(0, 0)
    start_scatter(0, 0)

    @pl.loop(0, (jnp.where(wid == NW - 1, EDGE_C_LAST, EDGE_C) - 4) // 4)
    def _(j):
        for p in range(4):
            k = 2 + 4 * j + p
            p2 = p % 2
            p4 = (2 + p) % 4
            wait_scatter(p2, p)
            start_idx(k + 2, p)
            wait_row(k, p4)
            start_gather(p4, p2)
            q2 = 1 - p2
            q4 = (p + 1) % 4
            wait_gather(q4, q2)
            wait_col(k - 1, q4)
            start_scatter(q2, q4)

    wait_scatter(0, 0)
    wait_row(n_chunks - 2, 2)
    start_gather(2, 0)
    wait_gather(1, 1)
    wait_col(n_chunks - 3, 1)
    start_scatter(1, 1)
    wait_scatter(1, 1)
    wait_row(n_chunks - 1, 3)
    start_gather(3, 1)
    wait_gather(2, 0)
    wait_col(n_chunks - 2, 2)
    start_scatter(0, 2)
    wait_gather(3, 1)
    wait_col(n_chunks - 1, 3)
    start_scatter(1, 3)
    wait_scatter(0, 2)
    wait_scatter(1, 3)

    plsc.subcore_barrier()
    pltpu.sync_copy(acc_sh.at[pl.ds(r0, ROWS_PER_TILE)],
                    out_hbm.at[pl.ds(cid * N_PAD + r0, ROWS_PER_TILE)])


_BLK = 640
_GRID = -(-N // _BLK)


def _linear_body(x_ref, w_ref, b_ref, dega_ref, degb_ref, o_ref):
    h = lax.dot_general(
        x_ref[...], w_ref[...], (((1,), (1,)), ((), ())),
        preferred_element_type=jnp.float32) + b_ref[...]
    deg = dega_ref[...] + degb_ref[...] + 1.0
    o_ref[...] = h * lax.rsqrt(deg)


def _tc_linear_scale(x, W, b, deg_col):
    return pl.pallas_call(
        _linear_body,
        grid=(_GRID,),
        in_specs=[
            pl.BlockSpec((_BLK, D), lambda i: (i, 0)),
            pl.BlockSpec((D, D), lambda i: (0, 0)),
            pl.BlockSpec((1, D), lambda i: (0, 0)),
            pl.BlockSpec((_BLK, 1), lambda i: (i, 0)),
            pl.BlockSpec((_BLK, 1), lambda i: (i + N_PAD // _BLK, 0)),
        ],
        out_specs=pl.BlockSpec((_BLK, D), lambda i: (i, 0)),
        out_shape=jax.ShapeDtypeStruct((N, D), jnp.float32),
    )(x, W, b.reshape(1, D), deg_col, deg_col)


def _final_body(s0_ref, s1_ref, g_ref, dega_ref, degb_ref, x_ref,
                gamma_ref, beta_ref, o_ref):
    deg = dega_ref[...] + degb_ref[...] + 1.0
    dinv = lax.rsqrt(deg)
    t = dinv * (s0_ref[...] + s1_ref[...] + g_ref[...])
    t = jnp.maximum(t, 0.0)
    mu = jnp.mean(t, axis=-1, keepdims=True)
    var = jnp.mean((t - mu) ** 2, axis=-1, keepdims=True)
    y = (t - mu) * lax.rsqrt(var + 1e-5)
    o_ref[...] = y * gamma_ref[...] + beta_ref[...] + x_ref[...]


def _tc_final(s_p, g, deg_p, x, gamma, beta):
    return pl.pallas_call(
        _final_body,
        grid=(_GRID,),
        in_specs=[
            pl.BlockSpec((_BLK, D), lambda i: (i, 0)),
            pl.BlockSpec((_BLK, D), lambda i: (i + N_PAD // _BLK, 0)),
            pl.BlockSpec((_BLK, D), lambda i: (i, 0)),
            pl.BlockSpec((_BLK, 1), lambda i: (i, 0)),
            pl.BlockSpec((_BLK, 1), lambda i: (i + N_PAD // _BLK, 0)),
            pl.BlockSpec((_BLK, D), lambda i: (i, 0)),
            pl.BlockSpec((1, D), lambda i: (0, 0)),
            pl.BlockSpec((1, D), lambda i: (0, 0)),
        ],
        out_specs=pl.BlockSpec((_BLK, D), lambda i: (i, 0)),
        out_shape=jax.ShapeDtypeStruct((N, D), jnp.float32),
    )(s_p, s_p, g, deg_p, deg_p, x, gamma.reshape(1, D), beta.reshape(1, D))


def kernel(x, edge_index, W, b, gamma, beta):
    row = edge_index[0].astype(jnp.int32)
    col = edge_index[1].astype(jnp.int32)

    zeros_deg = jnp.zeros((N_PAD,), jnp.float32)
    ones_upd = jnp.ones((DEG_B,), jnp.float32)
    zeros_acc = jnp.zeros((N_PAD, D), jnp.float32)

    deg_p = _sc_degree(row, zeros_deg, ones_upd)
    deg_col = deg_p.reshape(2 * N_PAD, 1)
    g = _tc_linear_scale(x, W, b, deg_col)
    s_p = _sc_edges(row, col, g, zeros_acc)
    return _tc_final(s_p, g, deg_col, x, gamma, beta)


# speedup vs baseline: 1.0745x; 1.0435x over previous
import functools

import jax
import jax.numpy as jnp
from jax import lax
from jax.experimental import pallas as pl
from jax.experimental.pallas import tpu as pltpu
from jax.experimental.pallas import tpu_sc as plsc

N = 10000
E = 320000
D = 128

NC = 2
NS = 16
NW = NC * NS
N_PAD = 10240
ROWS_PER_TILE = N_PAD // NS

EDGE_B = 160
EDGE_C = 64
EDGE_C_LAST = 16
E_PER_W = EDGE_B * EDGE_C

DEG_B = 2000
DEG_E_PER_W = E // NW
DEG_CHUNKS = DEG_E_PER_W // DEG_B

_mesh = plsc.VectorSubcoreMesh(core_axis_name="c", subcore_axis_name="s")


@functools.partial(
    pl.kernel,
    out_type=jax.ShapeDtypeStruct((2 * N_PAD,), jnp.float32),
    mesh=_mesh,
    scratch_types=[
        pltpu.VMEM((DEG_B,), jnp.int32),
        pltpu.VMEM((DEG_B,), jnp.float32),
        pltpu.VMEM_SHARED((N_PAD,), jnp.float32),
    ],
)
def _sc_degree(row_hbm, zeros_hbm, ones_hbm, out_hbm, idx_v, ones_v, deg_sh):
    cid = lax.axis_index("c")
    sid = lax.axis_index("s")
    wid = sid * NC + cid
    r0 = sid * ROWS_PER_TILE
    pltpu.sync_copy(zeros_hbm, deg_sh.at[pl.ds(r0, ROWS_PER_TILE)])
    pltpu.sync_copy(ones_hbm, ones_v)
    plsc.subcore_barrier()

    base = wid * DEG_E_PER_W

    @pl.loop(0, DEG_CHUNKS)
    def _(k):
        pltpu.sync_copy(row_hbm.at[pl.ds(base + k * DEG_B, DEG_B)], idx_v)
        pltpu.sync_copy(ones_v, deg_sh.at[idx_v], add=True)

    plsc.subcore_barrier()
    pltpu.sync_copy(deg_sh.at[pl.ds(r0, ROWS_PER_TILE)],
                    out_hbm.at[pl.ds(cid * N_PAD + r0, ROWS_PER_TILE)])


@functools.partial(
    pl.kernel,
    out_type=jax.ShapeDtypeStruct((2 * N_PAD, D), jnp.float32),
    mesh=_mesh,
    scratch_types=[
        pltpu.VMEM((EDGE_B,), jnp.int32),
        pltpu.VMEM((EDGE_B,), jnp.int32),
        pltpu.VMEM((EDGE_B,), jnp.int32),
        pltpu.VMEM((EDGE_B,), jnp.int32),
        pltpu.VMEM((EDGE_B,), jnp.int32),
        pltpu.VMEM((EDGE_B,), jnp.int32),
        pltpu.VMEM((EDGE_B,), jnp.int32),
        pltpu.VMEM((EDGE_B,), jnp.int32),
        pltpu.VMEM((EDGE_B, D), jnp.float32),
        pltpu.VMEM((EDGE_B, D), jnp.float32),
        pltpu.VMEM_SHARED((N_PAD, D), jnp.float32),
        pltpu.SemaphoreType.DMA,
        pltpu.SemaphoreType.DMA,
        pltpu.SemaphoreType.DMA,
        pltpu.SemaphoreType.DMA,
        pltpu.SemaphoreType.DMA,
        pltpu.SemaphoreType.DMA,
        pltpu.SemaphoreType.DMA,
        pltpu.SemaphoreType.DMA,
        pltpu.SemaphoreType.DMA,
        pltpu.SemaphoreType.DMA,
        pltpu.SemaphoreType.DMA,
        pltpu.SemaphoreType.DMA,
    ],
)
def _sc_edges(row_hbm, col_hbm, g_hbm, zeros_hbm, out_hbm,
              rb0, rb1, rb2, rb3, cb0, cb1, cb2, cb3, rows0, rows1, acc_sh,
              sr0, sr1, sr2, sr3, sc0, sc1, sc2, sc3, sg0, sg1, ss0, ss1):
    cid = lax.axis_index("c")
    sid = lax.axis_index("s")
    wid = sid * NC + cid
    r0 = sid * ROWS_PER_TILE
    base = wid * E_PER_W
    n_chunks = jnp.where(wid == NW - 1, EDGE_C_LAST, EDGE_C)
    rb = (rb0, rb1, rb2, rb3)
    cb = (cb0, cb1, cb2, cb3)
    rows = (rows0, rows1)
    sr = (sr0, sr1, sr2, sr3)
    sc = (sc0, sc1, sc2, sc3)
    sg = (sg0, sg1)
    ss = (ss0, ss1)

    def start_idx(k, p4):
        off = base + k * EDGE_B
        pltpu.async_copy(row_hbm.at[pl.ds(off, EDGE_B)], rb[p4], sr[p4])
        pltpu.async_copy(col_hbm.at[pl.ds(off, EDGE_B)], cb[p4], sc[p4])

    def wait_row(k, p4):
        pltpu.make_async_copy(
            row_hbm.at[pl.ds(base + k * EDGE_B, EDGE_B)], rb[p4], sr[p4]).wait()

    def wait_col(k, p4):
        pltpu.make_async_copy(
            col_hbm.at[pl.ds(base + k * EDGE_B, EDGE_B)], cb[p4], sc[p4]).wait()

    def start_gather(p4, p2):
        return pltpu.async_copy(g_hbm.at[rb[p4]], rows[p2], sg[p2])

    def wait_gather(p4, p2):
        pltpu.make_async_copy(g_hbm.at[rb[p4]], rows[p2], sg[p2]).wait()

    def start_scatter(p2, p4):
        return pltpu.async_copy(rows[p2], acc_sh.at[cb[p4]], ss[p2], add=True)

    def wait_scatter(p2, p4):
        pltpu.make_async_copy(rows[p2], acc_sh.at[cb[p4]], ss[p2]).wait()

    start_idx(0, 0)
    start_idx(1, 1)
    start_idx(2, 2)
    start_idx(3, 3)
    wait_row(0, 0)
    start_gather(0, 0)
    wait_row(1, 1)
    start_gather(1, 1)
    pltpu.sync_copy(zeros_hbm, acc_sh.at[pl.ds(r0, ROWS_PER_TILE)])
    plsc.subcore_barrier()
    wait_gather(0, 0)
    wait_col(0, 0)
    start_scatter(0, 0)

    @pl.loop(0, (jnp.where(wid == NW - 1, EDGE_C_LAST, EDGE_C) - 4) // 4)
    def _(j):
        for p in range(4):
            k = 2 + 4 * j + p
            p2 = p % 2
            p4 = (2 + p) % 4
            wait_scatter(p2, p)
            start_idx(k + 2, p)
            wait_row(k, p4)
            start_gather(p4, p2)
            q2 = 1 - p2
            q4 = (p + 1) % 4
            wait_gather(q4, q2)
            wait_col(k - 1, q4)
            start_scatter(q2, q4)

    wait_scatter(0, 0)
    wait_row(n_chunks - 2, 2)
    start_gather(2, 0)
    wait_gather(1, 1)
    wait_col(n_chunks - 3, 1)
    start_scatter(1, 1)
    wait_scatter(1, 1)
    wait_row(n_chunks - 1, 3)
    start_gather(3, 1)
    wait_gather(2, 0)
    wait_col(n_chunks - 2, 2)
    start_scatter(0, 2)
    wait_gather(3, 1)
    wait_col(n_chunks - 1, 3)
    start_scatter(1, 3)
    wait_scatter(0, 2)
    wait_scatter(1, 3)

    plsc.subcore_barrier()
    pltpu.sync_copy(acc_sh.at[pl.ds(r0, ROWS_PER_TILE)],
                    out_hbm.at[pl.ds(cid * N_PAD + r0, ROWS_PER_TILE)])


_BLK = 2048
_GRID = -(-N // _BLK)


def _linear_body(x_ref, w_ref, b_ref, o_ref):
    o_ref[...] = lax.dot_general(
        x_ref[...], w_ref[...], (((1,), (1,)), ((), ())),
        preferred_element_type=jnp.float32) + b_ref[...]


def _tc_linear(x, W, b):
    return pl.pallas_call(
        _linear_body,
        grid=(_GRID,),
        in_specs=[
            pl.BlockSpec((_BLK, D), lambda i: (i, 0)),
            pl.BlockSpec((D, D), lambda i: (0, 0)),
            pl.BlockSpec((1, D), lambda i: (0, 0)),
        ],
        out_specs=pl.BlockSpec((_BLK, D), lambda i: (i, 0)),
        out_shape=jax.ShapeDtypeStruct((N, D), jnp.float32),
    )(x, W, b.reshape(1, D))


def _scale_body(h_ref, dega_ref, degb_ref, o_ref):
    deg = dega_ref[...] + degb_ref[...] + 1.0
    o_ref[...] = h_ref[...] * lax.rsqrt(deg)


def _tc_scale(h, deg_col):
    return pl.pallas_call(
        _scale_body,
        grid=(_GRID,),
        in_specs=[
            pl.BlockSpec((_BLK, D), lambda i: (i, 0)),
            pl.BlockSpec((_BLK, 1), lambda i: (i, 0)),
            pl.BlockSpec((_BLK, 1), lambda i: (i + N_PAD // _BLK, 0)),
        ],
        out_specs=pl.BlockSpec((_BLK, D), lambda i: (i, 0)),
        out_shape=jax.ShapeDtypeStruct((N, D), jnp.float32),
    )(h, deg_col, deg_col)


def _final_body(s0_ref, s1_ref, g_ref, dega_ref, degb_ref, x_ref,
                gamma_ref, beta_ref, o_ref):
    deg = dega_ref[...] + degb_ref[...] + 1.0
    dinv = lax.rsqrt(deg)
    t = dinv * (s0_ref[...] + s1_ref[...] + g_ref[...])
    t = jnp.maximum(t, 0.0)
    mu = jnp.mean(t, axis=-1, keepdims=True)
    var = jnp.mean((t - mu) ** 2, axis=-1, keepdims=True)
    y = (t - mu) * lax.rsqrt(var + 1e-5)
    o_ref[...] = y * gamma_ref[...] + beta_ref[...] + x_ref[...]


def _tc_final(s_p, g, deg_p, x, gamma, beta):
    return pl.pallas_call(
        _final_body,
        grid=(_GRID,),
        in_specs=[
            pl.BlockSpec((_BLK, D), lambda i: (i, 0)),
            pl.BlockSpec((_BLK, D), lambda i: (i + N_PAD // _BLK, 0)),
            pl.BlockSpec((_BLK, D), lambda i: (i, 0)),
            pl.BlockSpec((_BLK, 1), lambda i: (i, 0)),
            pl.BlockSpec((_BLK, 1), lambda i: (i + N_PAD // _BLK, 0)),
            pl.BlockSpec((_BLK, D), lambda i: (i, 0)),
            pl.BlockSpec((1, D), lambda i: (0, 0)),
            pl.BlockSpec((1, D), lambda i: (0, 0)),
        ],
        out_specs=pl.BlockSpec((_BLK, D), lambda i: (i, 0)),
        out_shape=jax.ShapeDtypeStruct((N, D), jnp.float32),
    )(s_p, s_p, g, deg_p, deg_p, x, gamma.reshape(1, D), beta.reshape(1, D))


def kernel(x, edge_index, W, b, gamma, beta):
    row = edge_index[0].astype(jnp.int32)
    col = edge_index[1].astype(jnp.int32)

    zeros_deg = jnp.zeros((ROWS_PER_TILE,), jnp.float32)
    ones_upd = jnp.ones((DEG_B,), jnp.float32)
    zeros_acc = jnp.zeros((ROWS_PER_TILE, D), jnp.float32)

    deg_p = _sc_degree(row, zeros_deg, ones_upd)
    h = _tc_linear(x, W, b)
    deg_col = deg_p.reshape(2 * N_PAD, 1)
    g = _tc_scale(h, deg_col)
    s_p = _sc_edges(row, col, g, zeros_acc)
    return _tc_final(s_p, g, deg_col, x, gamma, beta)


# speedup vs baseline: 1.0766x; 1.0019x over previous
import functools

import jax
import jax.numpy as jnp
from jax import lax
from jax.experimental import pallas as pl
from jax.experimental.pallas import tpu as pltpu
from jax.experimental.pallas import tpu_sc as plsc

N = 10000
E = 320000
D = 128

NC = 2
NS = 16
NW = NC * NS
N_PAD = 10240
ROWS_PER_TILE = N_PAD // NS

EDGE_B = 160
EDGE_C = 64
EDGE_C_LAST = 16
E_PER_W = EDGE_B * EDGE_C

DEG_B = 10000
DEG_E_PER_W = E // NW
DEG_CHUNKS = DEG_E_PER_W // DEG_B

_mesh = plsc.VectorSubcoreMesh(core_axis_name="c", subcore_axis_name="s")


@functools.partial(
    pl.kernel,
    out_type=jax.ShapeDtypeStruct((2 * N_PAD,), jnp.float32),
    mesh=_mesh,
    scratch_types=[
        pltpu.VMEM((DEG_B,), jnp.int32),
        pltpu.VMEM((DEG_B,), jnp.float32),
        pltpu.VMEM_SHARED((N_PAD,), jnp.float32),
    ],
)
def _sc_degree(row_hbm, zeros_hbm, ones_hbm, out_hbm, idx_v, ones_v, deg_sh):
    cid = lax.axis_index("c")
    sid = lax.axis_index("s")
    wid = sid * NC + cid
    r0 = sid * ROWS_PER_TILE
    pltpu.sync_copy(zeros_hbm, deg_sh.at[pl.ds(r0, ROWS_PER_TILE)])
    pltpu.sync_copy(ones_hbm, ones_v)
    plsc.subcore_barrier()

    base = wid * DEG_E_PER_W

    @pl.loop(0, DEG_CHUNKS)
    def _(k):
        pltpu.sync_copy(row_hbm.at[pl.ds(base + k * DEG_B, DEG_B)], idx_v)
        pltpu.sync_copy(ones_v, deg_sh.at[idx_v], add=True)

    plsc.subcore_barrier()
    pltpu.sync_copy(deg_sh.at[pl.ds(r0, ROWS_PER_TILE)],
                    out_hbm.at[pl.ds(cid * N_PAD + r0, ROWS_PER_TILE)])


@functools.partial(
    pl.kernel,
    out_type=jax.ShapeDtypeStruct((2 * N_PAD, D), jnp.float32),
    mesh=_mesh,
    scratch_types=[
        pltpu.VMEM((EDGE_B,), jnp.int32),
        pltpu.VMEM((EDGE_B,), jnp.int32),
        pltpu.VMEM((EDGE_B,), jnp.int32),
        pltpu.VMEM((EDGE_B,), jnp.int32),
        pltpu.VMEM((EDGE_B,), jnp.int32),
        pltpu.VMEM((EDGE_B,), jnp.int32),
        pltpu.VMEM((EDGE_B,), jnp.int32),
        pltpu.VMEM((EDGE_B,), jnp.int32),
        pltpu.VMEM((EDGE_B, D), jnp.float32),
        pltpu.VMEM((EDGE_B, D), jnp.float32),
        pltpu.VMEM_SHARED((N_PAD, D), jnp.float32),
        pltpu.SemaphoreType.DMA,
        pltpu.SemaphoreType.DMA,
        pltpu.SemaphoreType.DMA,
        pltpu.SemaphoreType.DMA,
        pltpu.SemaphoreType.DMA,
        pltpu.SemaphoreType.DMA,
        pltpu.SemaphoreType.DMA,
        pltpu.SemaphoreType.DMA,
        pltpu.SemaphoreType.DMA,
        pltpu.SemaphoreType.DMA,
        pltpu.SemaphoreType.DMA,
        pltpu.SemaphoreType.DMA,
    ],
)
def _sc_edges(row_hbm, col_hbm, g_hbm, zeros_hbm, out_hbm,
              rb0, rb1, rb2, rb3, cb0, cb1, cb2, cb3, rows0, rows1, acc_sh,
              sr0, sr1, sr2, sr3, sc0, sc1, sc2, sc3, sg0, sg1, ss0, ss1):
    cid = lax.axis_index("c")
    sid = lax.axis_index("s")
    wid = sid * NC + cid
    r0 = sid * ROWS_PER_TILE
    base = wid * E_PER_W
    n_chunks = jnp.where(wid == NW - 1, EDGE_C_LAST, EDGE_C)
    rb = (rb0, rb1, rb2, rb3)
    cb = (cb0, cb1, cb2, cb3)
    rows = (rows0, rows1)
    sr = (sr0, sr1, sr2, sr3)
    sc = (sc0, sc1, sc2, sc3)
    sg = (sg0, sg1)
    ss = (ss0, ss1)

    def start_idx(k, p4):
        off = base + k * EDGE_B
        pltpu.async_copy(row_hbm.at[pl.ds(off, EDGE_B)], rb[p4], sr[p4])
        pltpu.async_copy(col_hbm.at[pl.ds(off, EDGE_B)], cb[p4], sc[p4])

    def wait_row(k, p4):
        pltpu.make_async_copy(
            row_hbm.at[pl.ds(base + k * EDGE_B, EDGE_B)], rb[p4], sr[p4]).wait()

    def wait_col(k, p4):
        pltpu.make_async_copy(
            col_hbm.at[pl.ds(base + k * EDGE_B, EDGE_B)], cb[p4], sc[p4]).wait()

    def start_gather(p4, p2):
        return pltpu.async_copy(g_hbm.at[rb[p4]], rows[p2], sg[p2])

    def wait_gather(p4, p2):
        pltpu.make_async_copy(g_hbm.at[rb[p4]], rows[p2], sg[p2]).wait()

    def start_scatter(p2, p4):
        return pltpu.async_copy(rows[p2], acc_sh.at[cb[p4]], ss[p2], add=True)

    def wait_scatter(p2, p4):
        pltpu.make_async_copy(rows[p2], acc_sh.at[cb[p4]], ss[p2]).wait()

    start_idx(0, 0)
    start_idx(1, 1)
    start_idx(2, 2)
    start_idx(3, 3)
    wait_row(0, 0)
    start_gather(0, 0)
    wait_row(1, 1)
    start_gather(1, 1)
    pltpu.sync_copy(zeros_hbm, acc_sh.at[pl.ds(r0, ROWS_PER_TILE)])
    plsc.subcore_barrier()
    wait_gather(0, 0)
    wait_col(0, 0)
    start_scatter(0, 0)

    @pl.loop(0, (jnp.where(wid == NW - 1, EDGE_C_LAST, EDGE_C) - 4) // 4)
    def _(j):
        for p in range(4):
            k = 2 + 4 * j + p
            p2 = p % 2
            p4 = (2 + p) % 4
            wait_scatter(p2, p)
            start_idx(k + 2, p)
            wait_row(k, p4)
            start_gather(p4, p2)
            q2 = 1 - p2
            q4 = (p + 1) % 4
            wait_gather(q4, q2)
            wait_col(k - 1, q4)
            start_scatter(q2, q4)

    wait_scatter(0, 0)
    wait_row(n_chunks - 2, 2)
    start_gather(2, 0)
    wait_gather(1, 1)
    wait_col(n_chunks - 3, 1)
    start_scatter(1, 1)
    wait_scatter(1, 1)
    wait_row(n_chunks - 1, 3)
    start_gather(3, 1)
    wait_gather(2, 0)
    wait_col(n_chunks - 2, 2)
    start_scatter(0, 2)
    wait_gather(3, 1)
    wait_col(n_chunks - 1, 3)
    start_scatter(1, 3)
    wait_scatter(0, 2)
    wait_scatter(1, 3)

    plsc.subcore_barrier()
    pltpu.sync_copy(acc_sh.at[pl.ds(r0, ROWS_PER_TILE)],
                    out_hbm.at[pl.ds(cid * N_PAD + r0, ROWS_PER_TILE)])


_BLK = 2048
_GRID = -(-N // _BLK)


def _linear_body(x_ref, w_ref, b_ref, o_ref):
    o_ref[...] = lax.dot_general(
        x_ref[...], w_ref[...], (((1,), (1,)), ((), ())),
        preferred_element_type=jnp.float32) + b_ref[...]


def _tc_linear(x, W, b):
    return pl.pallas_call(
        _linear_body,
        grid=(_GRID,),
        in_specs=[
            pl.BlockSpec((_BLK, D), lambda i: (i, 0)),
            pl.BlockSpec((D, D), lambda i: (0, 0)),
            pl.BlockSpec((1, D), lambda i: (0, 0)),
        ],
        out_specs=pl.BlockSpec((_BLK, D), lambda i: (i, 0)),
        out_shape=jax.ShapeDtypeStruct((N, D), jnp.float32),
    )(x, W, b.reshape(1, D))


def _scale_body(h_ref, dega_ref, degb_ref, o_ref):
    deg = dega_ref[...] + degb_ref[...] + 1.0
    o_ref[...] = h_ref[...] * lax.rsqrt(deg)


def _tc_scale(h, deg_col):
    return pl.pallas_call(
        _scale_body,
        grid=(_GRID,),
        in_specs=[
            pl.BlockSpec((_BLK, D), lambda i: (i, 0)),
            pl.BlockSpec((_BLK, 1), lambda i: (i, 0)),
            pl.BlockSpec((_BLK, 1), lambda i: (i + N_PAD // _BLK, 0)),
        ],
        out_specs=pl.BlockSpec((_BLK, D), lambda i: (i, 0)),
        out_shape=jax.ShapeDtypeStruct((N, D), jnp.float32),
    )(h, deg_col, deg_col)


def _final_body(s0_ref, s1_ref, g_ref, dega_ref, degb_ref, x_ref,
                gamma_ref, beta_ref, o_ref):
    deg = dega_ref[...] + degb_ref[...] + 1.0
    dinv = lax.rsqrt(deg)
    t = dinv * (s0_ref[...] + s1_ref[...] + g_ref[...])
    t = jnp.maximum(t, 0.0)
    mu = jnp.mean(t, axis=-1, keepdims=True)
    var = jnp.mean((t - mu) ** 2, axis=-1, keepdims=True)
    y = (t - mu) * lax.rsqrt(var + 1e-5)
    o_ref[...] = y * gamma_ref[...] + beta_ref[...] + x_ref[...]


def _tc_final(s_p, g, deg_p, x, gamma, beta):
    return pl.pallas_call(
        _final_body,
        grid=(_GRID,),
        in_specs=[
            pl.BlockSpec((_BLK, D), lambda i: (i, 0)),
            pl.BlockSpec((_BLK, D), lambda i: (i + N_PAD // _BLK, 0)),
            pl.BlockSpec((_BLK, D), lambda i: (i, 0)),
            pl.BlockSpec((_BLK, 1), lambda i: (i, 0)),
            pl.BlockSpec((_BLK, 1), lambda i: (i + N_PAD // _BLK, 0)),
            pl.BlockSpec((_BLK, D), lambda i: (i, 0)),
            pl.BlockSpec((1, D), lambda i: (0, 0)),
            pl.BlockSpec((1, D), lambda i: (0, 0)),
        ],
        out_specs=pl.BlockSpec((_BLK, D), lambda i: (i, 0)),
        out_shape=jax.ShapeDtypeStruct((N, D), jnp.float32),
    )(s_p, s_p, g, deg_p, deg_p, x, gamma.reshape(1, D), beta.reshape(1, D))


def kernel(x, edge_index, W, b, gamma, beta):
    row = edge_index[0].astype(jnp.int32)
    col = edge_index[1].astype(jnp.int32)

    zeros_deg = jnp.zeros((ROWS_PER_TILE,), jnp.float32)
    ones_upd = jnp.ones((DEG_B,), jnp.float32)
    zeros_acc = jnp.zeros((ROWS_PER_TILE, D), jnp.float32)

    deg_p = _sc_degree(row, zeros_deg, ones_upd)
    h = _tc_linear(x, W, b)
    deg_col = deg_p.reshape(2 * N_PAD, 1)
    g = _tc_scale(h, deg_col)
    s_p = _sc_edges(row, col, g, zeros_acc)
    return _tc_final(s_p, g, deg_col, x, gamma, beta)
